# baseline (speedup 1.0000x reference)
import jax
import jax.numpy as jnp
from jax import lax
from jax.experimental import pallas as pl
from jax.experimental.pallas import tpu as pltpu

_MU = 0.02
_NBINS = 10
_ALPHA = 0.75
_N = 8388608
_COLS = 128
_ROWS = _N // _COLS
_BLK = 4096
_GRID = _ROWS // _BLK
_CH = 8
_NCH = _BLK // _CH


def _ghmr_body(p_ref, t_ref, out_ref, acc_ref):
    step = pl.program_id(0)

    @pl.when(step == 0)
    def _init():
        for k in range(2 * _NBINS + 1):
            acc_ref[k] = jnp.float32(0.0)

    zero = jnp.zeros((_CH, _COLS), jnp.float32)

    def chunk(i, carry):
        cnt, s, ov = carry
        p = p_ref[pl.ds(i * _CH, _CH), :]
        t = t_ref[pl.ds(i * _CH, _CH), :]
        d = jnp.abs(p - t)
        loss = jnp.where(d < _MU, (0.5 / _MU) * d * d, d - 0.5 * _MU)
        g = jnp.abs(jnp.tanh(p) - jnp.tanh(t))
        bf = jnp.trunc(g * _NBINS)
        cnt = list(cnt)
        s = list(s)
        for k in range(_NBINS - 1):
            m = bf == jnp.float32(k)
            cnt[k] = cnt[k] + jnp.where(m, 1.0, 0.0)
            s[k] = s[k] + jnp.where(m, loss, 0.0)
        m9 = bf >= jnp.float32(_NBINS - 1)
        cnt[9] = cnt[9] + jnp.where(m9, 1.0, 0.0)
        s[9] = s[9] + jnp.where(m9, loss, 0.0)
        ov = ov + jnp.where(bf >= jnp.float32(_NBINS), 1.0, 0.0)
        return tuple(cnt), tuple(s), ov

    init = (tuple(zero for _ in range(_NBINS)),
            tuple(zero for _ in range(_NBINS)), zero)
    cnt, s, ov = lax.fori_loop(0, _NCH, chunk, init)

    for k in range(_NBINS):
        acc_ref[k] += jnp.sum(cnt[k])
        acc_ref[_NBINS + k] += jnp.sum(s[k])
    acc_ref[2 * _NBINS] += jnp.sum(ov)

    @pl.when(step == _GRID - 1)
    def _finish():
        total = jnp.float32(0.0)
        for k in range(_NBINS):
            c = acc_ref[k]
            if k == _NBINS - 1:
                c = c - acc_ref[2 * _NBINS]
            c = jnp.maximum(c, 1.0)
            w = jnp.exp(-_ALPHA * jnp.log(c))
            total = total + w * acc_ref[_NBINS + k]
        out_ref[0] = total / _N


def kernel(pred, target):
    p2 = pred.reshape(_ROWS, _COLS)
    t2 = target.reshape(_ROWS, _COLS)
    out = pl.pallas_call(
        _ghmr_body,
        grid=(_GRID,),
        in_specs=[
            pl.BlockSpec((_BLK, _COLS), lambda i: (i, 0)),
            pl.BlockSpec((_BLK, _COLS), lambda i: (i, 0)),
        ],
        out_specs=pl.BlockSpec(memory_space=pltpu.SMEM),
        out_shape=jax.ShapeDtypeStruct((1,), jnp.float32),
        scratch_shapes=[pltpu.SMEM((2 * _NBINS + 1,), jnp.float32)],
    )(p2, t2)
    return out[0]


# speedup vs baseline: 1.9246x; 1.9246x over previous
import jax
import jax.numpy as jnp
from jax import lax
from jax.experimental import pallas as pl
from jax.experimental.pallas import tpu as pltpu

_MU = 0.02
_NBINS = 10
_ALPHA = 0.75
_N = 8388608
_COLS = 128
_ROWS = _N // _COLS
_BLK = 4096
_GRID = _ROWS // _BLK
_CH = 8
_UNROLL = 4
_NCH = _BLK // (_CH * _UNROLL)


def _ghmr_body(p_ref, t_ref, out_ref, acc_ref):
    step = pl.program_id(0)

    @pl.when(step == 0)
    def _init():
        for k in range(2 * _NBINS + 1):
            acc_ref[k] = jnp.float32(0.0)

    zero = jnp.zeros((_CH, _COLS), jnp.float32)

    def chunk(i, carry):
        cnt, s, ov = carry
        cnt = list(cnt)
        s = list(s)
        for u in range(_UNROLL):
            base = (i * _UNROLL + u) * _CH
            p = p_ref[pl.ds(base, _CH), :]
            t = t_ref[pl.ds(base, _CH), :]
            d = jnp.abs(p - t)
            loss = jnp.where(d < _MU, (0.5 / _MU) * d * d, d - 0.5 * _MU)
            g = jnp.abs(jnp.tanh(p) - jnp.tanh(t))
            bf = jnp.trunc(g * _NBINS)
            for k in range(_NBINS - 1):
                m = bf == jnp.float32(k)
                cnt[k] = cnt[k] + jnp.where(m, 1.0, 0.0)
                s[k] = s[k] + jnp.where(m, loss, 0.0)
            m9 = bf >= jnp.float32(_NBINS - 1)
            cnt[9] = cnt[9] + jnp.where(m9, 1.0, 0.0)
            s[9] = s[9] + jnp.where(m9, loss, 0.0)
            ov = ov + jnp.where(bf >= jnp.float32(_NBINS), 1.0, 0.0)
        return tuple(cnt), tuple(s), ov

    init = (tuple(zero for _ in range(_NBINS)),
            tuple(zero for _ in range(_NBINS)), zero)
    cnt, s, ov = lax.fori_loop(0, _NCH, chunk, init)

    for k in range(_NBINS):
        acc_ref[k] += jnp.sum(cnt[k])
        acc_ref[_NBINS + k] += jnp.sum(s[k])
    acc_ref[2 * _NBINS] += jnp.sum(ov)

    @pl.when(step == _GRID - 1)
    def _finish():
        total = jnp.float32(0.0)
        for k in range(_NBINS):
            c = acc_ref[k]
            if k == _NBINS - 1:
                c = c - acc_ref[2 * _NBINS]
            c = jnp.maximum(c, 1.0)
            w = jnp.exp(-_ALPHA * jnp.log(c))
            total = total + w * acc_ref[_NBINS + k]
        out_ref[0] = total / _N


def kernel(pred, target):
    p2 = pred.reshape(_ROWS, _COLS)
    t2 = target.reshape(_ROWS, _COLS)
    out = pl.pallas_call(
        _ghmr_body,
        grid=(_GRID,),
        in_specs=[
            pl.BlockSpec((_BLK, _COLS), lambda i: (i, 0)),
            pl.BlockSpec((_BLK, _COLS), lambda i: (i, 0)),
        ],
        out_specs=pl.BlockSpec(memory_space=pltpu.SMEM),
        out_shape=jax.ShapeDtypeStruct((1,), jnp.float32),
        scratch_shapes=[pltpu.SMEM((2 * _NBINS + 1,), jnp.float32)],
    )(p2, t2)
    return out[0]


# speedup vs baseline: 2.2298x; 1.1586x over previous
import jax
import jax.numpy as jnp
from jax import lax
from jax.experimental import pallas as pl
from jax.experimental.pallas import tpu as pltpu

_MU = 0.02
_NBINS = 10
_ALPHA = 0.75
_N = 8388608
_COLS = 128
_ROWS = _N // _COLS
_BLK = 4096
_GRID = _ROWS // _BLK
_CH = 8
_UNROLL = 8
_NCH = _BLK // (_CH * _UNROLL)


def _ghmr_body(p_ref, t_ref, out_ref, acc_ref):
    step = pl.program_id(0)

    @pl.when(step == 0)
    def _init():
        for k in range(2 * _NBINS + 1):
            acc_ref[k] = jnp.float32(0.0)

    zero = jnp.zeros((_CH, _COLS), jnp.float32)

    def chunk(i, carry):
        cnt, s, ov = carry
        cnt = list(cnt)
        s = list(s)
        for u in range(_UNROLL):
            base = (i * _UNROLL + u) * _CH
            p = p_ref[pl.ds(base, _CH), :]
            t = t_ref[pl.ds(base, _CH), :]
            d = jnp.abs(p - t)
            loss = jnp.where(d < _MU, (0.5 / _MU) * d * d, d - 0.5 * _MU)
            g = jnp.abs(jnp.tanh(p) - jnp.tanh(t))
            bf = jnp.trunc(g * _NBINS)
            for k in range(_NBINS - 1):
                m = bf == jnp.float32(k)
                cnt[k] = cnt[k] + jnp.where(m, 1.0, 0.0)
                s[k] = s[k] + jnp.where(m, loss, 0.0)
            m9 = bf >= jnp.float32(_NBINS - 1)
            cnt[9] = cnt[9] + jnp.where(m9, 1.0, 0.0)
            s[9] = s[9] + jnp.where(m9, loss, 0.0)
            ov = ov + jnp.where(bf >= jnp.float32(_NBINS), 1.0, 0.0)
        return tuple(cnt), tuple(s), ov

    init = (tuple(zero for _ in range(_NBINS)),
            tuple(zero for _ in range(_NBINS)), zero)
    cnt, s, ov = lax.fori_loop(0, _NCH, chunk, init)

    for k in range(_NBINS):
        acc_ref[k] += jnp.sum(cnt[k])
        acc_ref[_NBINS + k] += jnp.sum(s[k])
    acc_ref[2 * _NBINS] += jnp.sum(ov)

    @pl.when(step == _GRID - 1)
    def _finish():
        total = jnp.float32(0.0)
        for k in range(_NBINS):
            c = acc_ref[k]
            if k == _NBINS - 1:
                c = c - acc_ref[2 * _NBINS]
            c = jnp.maximum(c, 1.0)
            w = jnp.exp(-_ALPHA * jnp.log(c))
            total = total + w * acc_ref[_NBINS + k]
        out_ref[0] = total / _N


def kernel(pred, target):
    p2 = pred.reshape(_ROWS, _COLS)
    t2 = target.reshape(_ROWS, _COLS)
    out = pl.pallas_call(
        _ghmr_body,
        grid=(_GRID,),
        in_specs=[
            pl.BlockSpec((_BLK, _COLS), lambda i: (i, 0)),
            pl.BlockSpec((_BLK, _COLS), lambda i: (i, 0)),
        ],
        out_specs=pl.BlockSpec(memory_space=pltpu.SMEM),
        out_shape=jax.ShapeDtypeStruct((1,), jnp.float32),
        scratch_shapes=[pltpu.SMEM((2 * _NBINS + 1,), jnp.float32)],
    )(p2, t2)
    return out[0]


# speedup vs baseline: 2.3222x; 1.0414x over previous
import jax
import jax.numpy as jnp
from jax import lax
from jax.experimental import pallas as pl
from jax.experimental.pallas import tpu as pltpu

_MU = 0.02
_NBINS = 10
_ALPHA = 0.75
_N = 8388608
_COLS = 128
_ROWS = _N // _COLS
_BLK = 4096
_GRID = _ROWS // _BLK
_CH = 8
_UNROLL = 16
_NCH = _BLK // (_CH * _UNROLL)


def _ghmr_body(p_ref, t_ref, out_ref, acc_ref):
    step = pl.program_id(0)

    @pl.when(step == 0)
    def _init():
        for k in range(2 * _NBINS + 1):
            acc_ref[k] = jnp.float32(0.0)

    zero = jnp.zeros((_CH, _COLS), jnp.float32)

    def chunk(i, carry):
        cnt, s, ov = carry
        cnt = list(cnt)
        s = list(s)
        for u in range(_UNROLL):
            base = (i * _UNROLL + u) * _CH
            p = p_ref[pl.ds(base, _CH), :]
            t = t_ref[pl.ds(base, _CH), :]
            d = jnp.abs(p - t)
            loss = jnp.where(d < _MU, (0.5 / _MU) * d * d, d - 0.5 * _MU)
            g = jnp.abs(jnp.tanh(p) - jnp.tanh(t))
            bf = jnp.trunc(g * _NBINS)
            for k in range(_NBINS - 1):
                m = bf == jnp.float32(k)
                cnt[k] = cnt[k] + jnp.where(m, 1.0, 0.0)
                s[k] = s[k] + jnp.where(m, loss, 0.0)
            m9 = bf >= jnp.float32(_NBINS - 1)
            cnt[9] = cnt[9] + jnp.where(m9, 1.0, 0.0)
            s[9] = s[9] + jnp.where(m9, loss, 0.0)
            ov = ov + jnp.where(bf >= jnp.float32(_NBINS), 1.0, 0.0)
        return tuple(cnt), tuple(s), ov

    init = (tuple(zero for _ in range(_NBINS)),
            tuple(zero for _ in range(_NBINS)), zero)
    cnt, s, ov = lax.fori_loop(0, _NCH, chunk, init)

    for k in range(_NBINS):
        acc_ref[k] += jnp.sum(cnt[k])
        acc_ref[_NBINS + k] += jnp.sum(s[k])
    acc_ref[2 * _NBINS] += jnp.sum(ov)

    @pl.when(step == _GRID - 1)
    def _finish():
        total = jnp.float32(0.0)
        for k in range(_NBINS):
            c = acc_ref[k]
            if k == _NBINS - 1:
                c = c - acc_ref[2 * _NBINS]
            c = jnp.maximum(c, 1.0)
            w = jnp.exp(-_ALPHA * jnp.log(c))
            total = total + w * acc_ref[_NBINS + k]
        out_ref[0] = total / _N


def kernel(pred, target):
    p2 = pred.reshape(_ROWS, _COLS)
    t2 = target.reshape(_ROWS, _COLS)
    out = pl.pallas_call(
        _ghmr_body,
        grid=(_GRID,),
        in_specs=[
            pl.BlockSpec((_BLK, _COLS), lambda i: (i, 0)),
            pl.BlockSpec((_BLK, _COLS), lambda i: (i, 0)),
        ],
        out_specs=pl.BlockSpec(memory_space=pltpu.SMEM),
        out_shape=jax.ShapeDtypeStruct((1,), jnp.float32),
        scratch_shapes=[pltpu.SMEM((2 * _NBINS + 1,), jnp.float32)],
    )(p2, t2)
    return out[0]
